# Initial kernel scaffold; baseline (speedup 1.0000x reference)
import jax
import jax.numpy as jnp
from jax.experimental import pallas as pl


def kernel(x, embedding):
    raise NotImplementedError("write your pallas kernel here")


# speedup vs baseline: 1.4092x; 1.4092x over previous
import functools

import jax
import jax.numpy as jnp
from jax import lax
from jax.experimental import pallas as pl
from jax.experimental.pallas import tpu as pltpu
from jax.experimental.pallas import tpu_sc as plsc

VOCAB = 1000
BATCH = 4096
SEQ = 20

_INFO = plsc.get_sparse_core_info()
_NC = _INFO.num_cores
_NS = _INFO.num_subcores
_NW = _NC * _NS

_B = BATCH * SEQ
_D = VOCAB
_PER_W = _B // _NW
_C = 64
_G = _PER_W // _C


def _make_kernel():
    mesh = plsc.VectorSubcoreMesh(core_axis_name="c", subcore_axis_name="s")

    @functools.partial(
        pl.kernel,
        mesh=mesh,
        out_type=jax.ShapeDtypeStruct((_B, _D), jnp.float32),
        scratch_types=[
            pltpu.VMEM((_G, _C), jnp.int32),
            pltpu.VMEM((_C, _D), jnp.float32),
            pltpu.SemaphoreType.DMA,
        ],
        compiler_params=pltpu.CompilerParams(use_tc_tiling_on_sc=False),
    )
    def body(x_hbm, table_hbm, out_hbm, idx_v, buf_v, gsem):
        wid = lax.axis_index("s") * _NC + lax.axis_index("c")
        base = wid * _PER_W
        pltpu.sync_copy(x_hbm.at[wid], idx_v)

        def chunk(g, carry):
            pltpu.async_copy(table_hbm.at[idx_v.at[g]], buf_v, gsem).wait()
            pltpu.sync_copy(buf_v, out_hbm.at[pl.ds(base + g * _C, _C)])
            return carry

        lax.fori_loop(0, _G, chunk, 0)

    return body


_kernel_call = _make_kernel()


def kernel(x, embedding):
    idx = x.reshape(-1).astype(jnp.int32).reshape(_NW, _G, _C)
    out = _kernel_call(idx, embedding)
    return out.reshape(BATCH, SEQ, _D)


# speedup vs baseline: 1.4225x; 1.0094x over previous
import functools

import jax
import jax.numpy as jnp
from jax import lax
from jax.experimental import pallas as pl
from jax.experimental.pallas import tpu as pltpu
from jax.experimental.pallas import tpu_sc as plsc

VOCAB = 1000
BATCH = 4096
SEQ = 20

_INFO = plsc.get_sparse_core_info()
_NC = _INFO.num_cores
_NS = _INFO.num_subcores
_NW = _NC * _NS

_B = BATCH * SEQ
_D = VOCAB
_PER_W = _B // _NW
_DEPTH = 4
_C = 32
_G = _PER_W // _C
_NGROUP = _G // _DEPTH


def _make_kernel():
    mesh = plsc.VectorSubcoreMesh(core_axis_name="c", subcore_axis_name="s")

    @functools.partial(
        pl.kernel,
        mesh=mesh,
        out_type=jax.ShapeDtypeStruct((_B, _D), jnp.float32),
        scratch_types=(
            [pltpu.VMEM((_G, _C), jnp.int32)]
            + [pltpu.VMEM((_C, _D), jnp.float32) for _ in range(_DEPTH)]
            + [pltpu.SemaphoreType.DMA for _ in range(2 * _DEPTH)]
        ),
        compiler_params=pltpu.CompilerParams(use_tc_tiling_on_sc=False),
    )
    def body(x_hbm, table_hbm, out_hbm, idx_v, *rest):
        bufs = rest[:_DEPTH]
        gsems = rest[_DEPTH:2 * _DEPTH]
        ssems = rest[2 * _DEPTH:]
        wid = lax.axis_index("s") * _NC + lax.axis_index("c")
        base = wid * _PER_W
        pltpu.sync_copy(x_hbm.at[wid], idx_v)

        def fire_gather(g, j):
            pltpu.async_copy(table_hbm.at[idx_v.at[g]], bufs[j], gsems[j])

        def wait_gather(g, j):
            pltpu.make_async_copy(
                table_hbm.at[idx_v.at[g]], bufs[j], gsems[j]).wait()

        def fire_scatter(g, j):
            pltpu.async_copy(
                bufs[j], out_hbm.at[pl.ds(base + g * _C, _C)], ssems[j])

        def wait_scatter(g, j):
            pltpu.make_async_copy(
                bufs[j], out_hbm.at[pl.ds(base + g * _C, _C)], ssems[j]).wait()

        for j in range(_DEPTH):
            fire_gather(j, j)

        def group(gg, carry):
            for j in range(_DEPTH):
                g = gg * _DEPTH + j
                wait_gather(g, j)
                fire_scatter(g, j)
            for j in range(_DEPTH):
                g = gg * _DEPTH + j
                wait_scatter(g, j)
                fire_gather(g + _DEPTH, j)
            return carry

        lax.fori_loop(0, _NGROUP - 1, group, 0)

        for j in range(_DEPTH):
            g = (_NGROUP - 1) * _DEPTH + j
            wait_gather(g, j)
            fire_scatter(g, j)
        for j in range(_DEPTH):
            g = (_NGROUP - 1) * _DEPTH + j
            wait_scatter(g, j)

    return body


_kernel_call = _make_kernel()


def kernel(x, embedding):
    idx = x.reshape(-1).astype(jnp.int32).reshape(_NW, _G, _C)
    out = _kernel_call(idx, embedding)
    return out.reshape(BATCH, SEQ, _D)


# speedup vs baseline: 1.4275x; 1.0035x over previous
import functools

import jax
import jax.numpy as jnp
from jax import lax
from jax.experimental import pallas as pl
from jax.experimental.pallas import tpu as pltpu
from jax.experimental.pallas import tpu_sc as plsc

VOCAB = 1000
BATCH = 4096
SEQ = 20

_INFO = plsc.get_sparse_core_info()
_NC = _INFO.num_cores
_NS = _INFO.num_subcores
_NW = _NC * _NS

_B = BATCH * SEQ
_D = VOCAB
_PER_W = _B // _NW
_DEPTH = 4
_C = 20
_G = _PER_W // _C
_NGROUP = _G // _DEPTH


def _make_kernel():
    mesh = plsc.VectorSubcoreMesh(core_axis_name="c", subcore_axis_name="s")

    @functools.partial(
        pl.kernel,
        mesh=mesh,
        out_type=jax.ShapeDtypeStruct((BATCH, SEQ, _D), jnp.float32),
        scratch_types=(
            [pltpu.VMEM((_G, _C), jnp.int32)]
            + [pltpu.VMEM((_C // SEQ, SEQ, _D), jnp.float32)
               for _ in range(_DEPTH)]
            + [pltpu.SemaphoreType.DMA for _ in range(2 * _DEPTH)]
        ),
        compiler_params=pltpu.CompilerParams(use_tc_tiling_on_sc=False),
    )
    def body(x_hbm, table_hbm, out_hbm, idx_v, *rest):
        bufs = rest[:_DEPTH]
        gsems = rest[_DEPTH:2 * _DEPTH]
        ssems = rest[2 * _DEPTH:]
        wid = lax.axis_index("s") * _NC + lax.axis_index("c")
        base = wid * _PER_W
        pltpu.sync_copy(x_hbm.at[wid], idx_v)

        def fire_gather(g, j):
            pltpu.async_copy(table_hbm.at[idx_v.at[g]], bufs[j].at[0],
                             gsems[j])

        def wait_gather(g, j):
            pltpu.make_async_copy(
                table_hbm.at[idx_v.at[g]], bufs[j].at[0], gsems[j]).wait()

        def fire_scatter(g, j):
            pltpu.async_copy(
                bufs[j],
                out_hbm.at[pl.ds((base + g * _C) // SEQ, _C // SEQ)],
                ssems[j])

        def wait_scatter(g, j):
            pltpu.make_async_copy(
                bufs[j],
                out_hbm.at[pl.ds((base + g * _C) // SEQ, _C // SEQ)],
                ssems[j]).wait()

        for j in range(_DEPTH):
            fire_gather(j, j)

        def group(gg, carry):
            for j in range(_DEPTH):
                g = gg * _DEPTH + j
                wait_gather(g, j)
                fire_scatter(g, j)
            for j in range(_DEPTH):
                g = gg * _DEPTH + j
                wait_scatter(g, j)
                fire_gather(g + _DEPTH, j)
            return carry

        lax.fori_loop(0, _NGROUP - 1, group, 0)

        for j in range(_DEPTH):
            g = (_NGROUP - 1) * _DEPTH + j
            wait_gather(g, j)
            fire_scatter(g, j)
        for j in range(_DEPTH):
            g = (_NGROUP - 1) * _DEPTH + j
            wait_scatter(g, j)

    return body


_kernel_call = _make_kernel()


def kernel(x, embedding):
    idx = x.reshape(-1).astype(jnp.int32).reshape(_NW, _G, _C)
    return _kernel_call(idx, embedding)


# speedup vs baseline: 2.2617x; 1.5843x over previous
import functools

import jax
import jax.numpy as jnp
from jax import lax
from jax.experimental import pallas as pl
from jax.experimental.pallas import tpu as pltpu
from jax.experimental.pallas import tpu_sc as plsc

VOCAB = 1000
BATCH = 4096
SEQ = 20

_INFO = plsc.get_sparse_core_info()
_NC = _INFO.num_cores
_NS = _INFO.num_subcores
_NW = _NC * _NS

_B = BATCH * SEQ
_D = VOCAB
_DP = 1024
_PER_W = _B // _NW
_DEPTH = 4
_C = 20
_G = _PER_W // _C
_NGROUP = _G // _DEPTH


def _make_kernel():
    mesh = plsc.VectorSubcoreMesh(core_axis_name="c", subcore_axis_name="s")

    @functools.partial(
        pl.kernel,
        mesh=mesh,
        out_type=jax.ShapeDtypeStruct((BATCH, SEQ, _DP), jnp.float32),
        scratch_types=(
            [pltpu.VMEM((_G, _C), jnp.int32)]
            + [pltpu.VMEM((_C // SEQ, SEQ, _DP), jnp.float32)
               for _ in range(_DEPTH)]
            + [pltpu.SemaphoreType.DMA for _ in range(2 * _DEPTH)]
        ),
    )
    def body(x_hbm, table_hbm, out_hbm, idx_v, *rest):
        bufs = rest[:_DEPTH]
        gsems = rest[_DEPTH:2 * _DEPTH]
        ssems = rest[2 * _DEPTH:]
        wid = lax.axis_index("s") * _NC + lax.axis_index("c")
        base = wid * _PER_W
        pltpu.sync_copy(x_hbm.at[wid], idx_v)

        def fire_gather(g, j):
            pltpu.async_copy(table_hbm.at[idx_v.at[g]], bufs[j].at[0],
                             gsems[j])

        def wait_gather(g, j):
            pltpu.make_async_copy(
                table_hbm.at[idx_v.at[g]], bufs[j].at[0], gsems[j]).wait()

        def _scatter_args(g, j):
            src = bufs[j]
            dst = out_hbm.at[pl.ds((base + g * _C) // SEQ, _C // SEQ)]
            return src, dst

        def fire_scatter(g, j):
            src, dst = _scatter_args(g, j)
            pltpu.async_copy(src, dst, ssems[j])

        def wait_scatter(g, j):
            src, dst = _scatter_args(g, j)
            pltpu.make_async_copy(src, dst, ssems[j]).wait()

        for j in range(_DEPTH):
            fire_gather(j, j)

        def group(gg, carry):
            for j in range(_DEPTH):
                g = gg * _DEPTH + j
                wait_gather(g, j)
                fire_scatter(g, j)
            for j in range(_DEPTH):
                g = gg * _DEPTH + j
                wait_scatter(g, j)
                fire_gather(g + _DEPTH, j)
            return carry

        lax.fori_loop(0, _NGROUP - 1, group, 0)

        for j in range(_DEPTH):
            g = (_NGROUP - 1) * _DEPTH + j
            wait_gather(g, j)
            fire_scatter(g, j)
        for j in range(_DEPTH):
            g = (_NGROUP - 1) * _DEPTH + j
            wait_scatter(g, j)

    return body


_kernel_call = _make_kernel()


def kernel(x, embedding):
    idx = x.reshape(-1).astype(jnp.int32).reshape(_NW, _G, _C)
    table = jnp.pad(embedding, ((0, 0), (0, _DP - _D)))
    out = _kernel_call(idx, table)
    return out[:, :, :_D]
